# baseline (speedup 1.0000x reference)
import functools

import jax
import jax.numpy as jnp
from jax import lax
from jax.experimental import pallas as pl
from jax.experimental.pallas import tpu as pltpu
from jax.experimental.pallas import tpu_sc as plsc

B = 16
R = 64
C = 256
N = 32768
HW = R * R
CHW = C // 2
SUB = 128
NSUB = N // SUB
NTILE = 16
KPT = NSUB // NTILE
ROWS_PT = HW // NTILE
NG = 4
GB = B // NG


def _make_sc_body(b0):
    def _sc_body(feats_hbm, zeros_hbm, comb_hbm, out_hbm,
                 fbuf, cbuf, ibuf, acc0, acc1, sem_w0, sem_w1, sem_p):
        accs = (acc0, acc1)
        sems = (sem_w0, sem_w1)
        tid = lax.axis_index("s")
        r0 = tid * ROWS_PT

        pend = []
        for k in range(KPT):
            chunk = k * NTILE + tid
            pend.append(pltpu.async_copy(
                comb_hbm.at[pl.ds(chunk * SUB, SUB)],
                cbuf.at[pl.ds(k * SUB, SUB)], sem_p))
        for w in range(2):
            pend.append(pltpu.async_copy(
                zeros_hbm, accs[w].at[pl.ds(r0, ROWS_PT)], sems[w]))
        for d in pend:
            d.wait()

        desc_w = [None, None]
        for i in range(2 * GB):
            w = i % 2
            bl = i // 2
            b = b0 + bl
            c0 = w * CHW
            acc = accs[w]

            if desc_w[w] is not None:
                desc_w[w].wait()
                desc_w[w] = None
            plsc.subcore_barrier()

            def sub_body(k, carry, acc=acc, c0=c0, b=b):
                base = k * SUB
                bfirst = cbuf[pl.ds(base, 16)][0] >> 16
                blast = cbuf[pl.ds(base + SUB - 16, 16)][15] >> 16

                @pl.when(jnp.logical_and(bfirst <= b, b <= blast))
                def _():
                    chunk = k * NTILE + tid
                    pltpu.sync_copy(
                        feats_hbm.at[pl.ds(chunk * SUB, SUB),
                                     pl.ds(c0, CHW)],
                        fbuf)
                    for j in range(SUB // 16):
                        vc = cbuf[pl.ds(base + j * 16, 16)]
                        ibuf[pl.ds(j * 16, 16)] = jnp.where(
                            (vc >> 16) == b, vc & 0xFFFF, HW + tid)
                    pltpu.sync_copy(fbuf, acc.at[ibuf], add=True)

                return carry

            lax.fori_loop(0, KPT, sub_body, 0)
            plsc.subcore_barrier()
            desc_w[w] = pltpu.async_copy(
                acc.at[pl.ds(r0, ROWS_PT)],
                out_hbm.at[bl, pl.ds(r0, ROWS_PT), pl.ds(c0, CHW)],
                sems[w])

        for d in desc_w:
            if d is not None:
                d.wait()

    return _sc_body


def _make_sc(b0):
    return functools.partial(
        pl.kernel,
        out_type=jax.ShapeDtypeStruct((GB, HW, C), jnp.float32),
        mesh=plsc.VectorSubcoreMesh(core_axis_name="c", subcore_axis_name="s",
                                    num_cores=1),
        scratch_types=[
            pltpu.VMEM((SUB, CHW), jnp.float32),
            pltpu.VMEM((KPT * SUB,), jnp.int32),
            pltpu.VMEM((SUB,), jnp.int32),
            pltpu.VMEM_SHARED((HW + NTILE, CHW), jnp.float32),
            pltpu.VMEM_SHARED((HW + NTILE, CHW), jnp.float32),
            pltpu.SemaphoreType.DMA,
            pltpu.SemaphoreType.DMA,
            pltpu.SemaphoreType.DMA,
        ],
    )(_make_sc_body(b0))


_scatters = [_make_sc(g * GB) for g in range(NG)]


def _t_body(cur_ref, eye_ref, out_ref, prev_scr):
    bb = pl.program_id(0)
    cur = cur_ref[0]
    prev = jnp.where(bb == 0, jnp.zeros_like(cur), prev_scr[...])
    delta = cur - prev
    t = lax.dot_general(eye_ref[...], delta, (((1,), (1,)), ((), ())),
                        preferred_element_type=jnp.float32)
    out_ref[0] = t.reshape(C, R, R)
    prev_scr[...] = cur


def _t_body_chained(cur_ref, prev_out_ref, eye_ref, out_ref, prev_scr):
    del prev_out_ref
    _t_body(cur_ref, eye_ref, out_ref, prev_scr)


def _make_tr(b0, chained):
    in_specs = [pl.BlockSpec((1, HW, C), lambda bl: (bl, 0, 0))]
    if chained:
        in_specs.append(pl.BlockSpec(memory_space=pl.ANY))
    in_specs.append(pl.BlockSpec((C, C), lambda bl: (0, 0)))
    return pl.pallas_call(
        _t_body_chained if chained else _t_body,
        grid=(GB,),
        in_specs=in_specs,
        out_specs=pl.BlockSpec((1, C, R, R),
                               lambda bl, b0=b0: (b0 + bl, 0, 0, 0)),
        out_shape=jax.ShapeDtypeStruct((B, C, R, R), jnp.float32),
        scratch_shapes=[pltpu.VMEM((HW, C), jnp.float32)],
        input_output_aliases={1: 0} if chained else {},
    )


_transposes = [_make_tr(g * GB, g > 0) for g in range(NG)]


@jax.jit
def kernel(feats, batch_idx, spatial_idx):
    comb = ((batch_idx.astype(jnp.int32) << 16)
            | spatial_idx.astype(jnp.int32))
    zeros = jnp.zeros((ROWS_PT, CHW), jnp.float32)
    eye = jnp.eye(C, dtype=jnp.float32)
    denses = [sc(feats, zeros, comb) for sc in _scatters]
    out = _transposes[0](denses[0], eye)
    for g in range(1, NG):
        out = _transposes[g](denses[g], out, eye)
    return out


# speedup vs baseline: 1.0635x; 1.0635x over previous
import functools

import jax
import jax.numpy as jnp
from jax import lax
from jax.experimental import pallas as pl
from jax.experimental.pallas import tpu as pltpu
from jax.experimental.pallas import tpu_sc as plsc

B = 16
R = 64
C = 256
N = 32768
HW = R * R
CHW = C // 2
SUB = 128
NSUB = N // SUB
NTILE = 16
KPT = NSUB // NTILE
ROWS_PT = HW // NTILE


def _sc_body(feats_hbm, zeros_hbm, comb_hbm, out_hbm,
             fbuf, cbuf, ibuf, acc0, acc1, sem_w0, sem_w1, sem_p):
    accs = (acc0, acc1)
    sems = (sem_w0, sem_w1)
    tid = lax.axis_index("s")
    r0 = tid * ROWS_PT

    pend = []
    for k in range(KPT):
        chunk = k * NTILE + tid
        pend.append(pltpu.async_copy(comb_hbm.at[pl.ds(chunk * SUB, SUB)],
                                     cbuf.at[pl.ds(k * SUB, SUB)], sem_p))
    for w in range(2):
        pend.append(pltpu.async_copy(zeros_hbm,
                                     accs[w].at[pl.ds(r0, ROWS_PT)], sems[w]))
    for d in pend:
        d.wait()

    desc_w = [None, None]
    for i in range(2 * B):
        w = i % 2
        b = i // 2
        c0 = w * CHW
        acc = accs[w]

        if desc_w[w] is not None:
            desc_w[w].wait()
            desc_w[w] = None
        plsc.subcore_barrier()

        def sub_body(k, carry, acc=acc, c0=c0, b=b):
            base = k * SUB
            bfirst = cbuf[pl.ds(base, 16)][0] >> 16
            blast = cbuf[pl.ds(base + SUB - 16, 16)][15] >> 16

            @pl.when(jnp.logical_and(bfirst <= b, b <= blast))
            def _():
                chunk = k * NTILE + tid
                pltpu.sync_copy(
                    feats_hbm.at[pl.ds(chunk * SUB, SUB), pl.ds(c0, CHW)],
                    fbuf)
                for j in range(SUB // 16):
                    vc = cbuf[pl.ds(base + j * 16, 16)]
                    ibuf[pl.ds(j * 16, 16)] = jnp.where(
                        (vc >> 16) == b, vc & 0xFFFF, HW + tid)
                pltpu.sync_copy(fbuf, acc.at[ibuf], add=True)

            return carry

        lax.fori_loop(0, KPT, sub_body, 0)
        plsc.subcore_barrier()
        desc_w[w] = pltpu.async_copy(
            acc.at[pl.ds(r0, ROWS_PT)],
            out_hbm.at[b, pl.ds(r0, ROWS_PT), pl.ds(c0, CHW)],
            sems[w])

    for d in desc_w:
        if d is not None:
            d.wait()


_scatter_sc = functools.partial(
    pl.kernel,
    out_type=jax.ShapeDtypeStruct((B, HW, C), jnp.float32),
    mesh=plsc.VectorSubcoreMesh(core_axis_name="c", subcore_axis_name="s",
                                num_cores=1),
    scratch_types=[
        pltpu.VMEM((SUB, CHW), jnp.float32),
        pltpu.VMEM((KPT * SUB,), jnp.int32),
        pltpu.VMEM((SUB,), jnp.int32),
        pltpu.VMEM_SHARED((HW + NTILE, CHW), jnp.float32),
        pltpu.VMEM_SHARED((HW + NTILE, CHW), jnp.float32),
        pltpu.SemaphoreType.DMA,
        pltpu.SemaphoreType.DMA,
        pltpu.SemaphoreType.DMA,
    ],
)(_sc_body)


TH = 512


def _t_body(cur_ref, out_ref, prev_scr):
    bb = pl.program_id(1)
    cur = cur_ref[0]
    prev = jnp.where(bb == 0, jnp.zeros_like(cur), prev_scr[...])
    out_ref[0] = (cur - prev).T
    prev_scr[...] = cur


_transpose_tc = pl.pallas_call(
    _t_body,
    grid=(HW // TH, B),
    in_specs=[pl.BlockSpec((1, TH, C), lambda j, b: (b, j, 0))],
    out_specs=pl.BlockSpec((1, C, TH), lambda j, b: (b, 0, j)),
    out_shape=jax.ShapeDtypeStruct((B, C, HW), jnp.float32),
    scratch_shapes=[pltpu.VMEM((TH, C), jnp.float32)],
)


@jax.jit
def kernel(feats, batch_idx, spatial_idx):
    comb = ((batch_idx.astype(jnp.int32) << 16)
            | spatial_idx.astype(jnp.int32))
    zeros = jnp.zeros((ROWS_PT, CHW), jnp.float32)
    dense = _scatter_sc(feats, zeros, comb)
    out = _transpose_tc(dense)
    return out.reshape(B, C, R, R)
